# Initial kernel scaffold; baseline (speedup 1.0000x reference)
import jax
import jax.numpy as jnp
from jax.experimental import pallas as pl


def kernel(candidates, lengths, scores, ngram_scores, backtrans_scores, qa_scores):
    raise NotImplementedError("write your pallas kernel here")


# speedup vs baseline: 4.0875x; 4.0875x over previous
import functools

import jax
import jax.numpy as jnp
from jax import lax
from jax.experimental import pallas as pl
from jax.experimental.pallas import tpu as pltpu
from jax.experimental.pallas import tpu_sc as plsc

PAD_ID = 0
B, N, L = 64, 2048, 32
LANES = 16
V = N // LANES
ROWS_PER_W = 2


def _sort_vreg(v, desc):
    s = jnp.sort(v)
    return jnp.where(desc, lax.rev(s, (0,)), s)


_GATHER_1D = lax.GatherDimensionNumbers(
    offset_dims=(), collapsed_slice_dims=(0,), start_index_map=(0,))


def _permute(v, idx):
    return lax.gather(v, idx[:, None], _GATHER_1D, slice_sizes=(1,),
                      mode=lax.GatherScatterMode.PROMISE_IN_BOUNDS)


def _butterfly(v, op):
    iota = lax.iota(jnp.int32, LANES)
    for d in (1, 2, 4, 8):
        v = op(v, _permute(v, iota ^ d))
    return v


@functools.lru_cache(maxsize=1)
def _build():
    info = plsc.get_sparse_core_info()
    nc = info.num_cores

    def body(ng_hbm, bt_hbm, nll_hbm, qa_hbm,
             idxpad_hbm, sorted_hbm,
             ng_v, bt_v, nll_v, qa_v, comb_v, idx_v):
        wid = lax.axis_index("s") * nc + lax.axis_index("c")
        row0 = wid * ROWS_PER_W

        for r in range(ROWS_PER_W):
            pltpu.sync_copy(ng_hbm.at[row0 + r], ng_v.at[r])
            pltpu.sync_copy(bt_hbm.at[row0 + r], bt_v.at[r])
            pltpu.sync_copy(nll_hbm.at[row0 + r], nll_v.at[r])
            pltpu.sync_copy(qa_hbm.at[row0 + r], qa_v.at[r])

        def combine_body(j, carry):
            m0, m1 = carry
            sl = pl.ds(j * LANES, LANES)
            out = []
            for r in range(ROWS_PER_W):
                c = (ng_v[r, sl] * 1.5 + (bt_v[r, sl] + nll_v[r, sl]) * 0.5) \
                    * (qa_v[r, sl] * 0.9 + 0.1)
                comb_v[r, sl] = c
                out.append(c)
            return jnp.maximum(m0, out[0]), jnp.maximum(m1, out[1])

        minit = jnp.full((LANES,), -jnp.inf, jnp.float32)
        m0, m1 = lax.fori_loop(0, V, combine_body, (minit, minit))
        row_max = (_butterfly(m0, jnp.maximum), _butterfly(m1, jnp.maximum))

        def argmax_base_body(j, carry):
            i0, i1 = carry
            sl = pl.ds(j * LANES, LANES)
            iota = lax.iota(jnp.int32, LANES) + j * LANES
            desc = (j & 1) == 0
            idxs = []
            for r in range(ROWS_PER_W):
                c = comb_v[r, sl]
                idxs.append(jnp.where(c == row_max[r], iota, N))
                comb_v[r, sl] = _sort_vreg(c, desc)
            return jnp.minimum(i0, idxs[0]), jnp.minimum(i1, idxs[1])

        iinit = jnp.full((LANES,), N, jnp.int32)
        i0v, i1v = lax.fori_loop(0, V, argmax_base_body, (iinit, iinit))
        top_idx = (_butterfly(i0v, jnp.minimum), _butterfly(i1v, jnp.minimum))

        def inter_pass(w, d):
            def pass_body(j, _):
                a = (j // d) * (2 * d) + (j % d)
                b = a + d
                desc = ((a // w) & 1) == 0
                sa = pl.ds(a * LANES, LANES)
                sb = pl.ds(b * LANES, LANES)
                for r in range(ROWS_PER_W):
                    va = comb_v[r, sa]
                    vb = comb_v[r, sb]
                    hi = jnp.maximum(va, vb)
                    lo = jnp.minimum(va, vb)
                    comb_v[r, sa] = jnp.where(desc, hi, lo)
                    comb_v[r, sb] = jnp.where(desc, lo, hi)
                return 0

            lax.fori_loop(0, V // 2, pass_body, 0)

        def intra_pass(w):
            def pass_body(i, _):
                desc = ((i // w) & 1) == 0
                sl = pl.ds(i * LANES, LANES)
                for r in range(ROWS_PER_W):
                    comb_v[r, sl] = _sort_vreg(comb_v[r, sl], desc)
                return 0

            lax.fori_loop(0, V, pass_body, 0)

        for k in range(5, 12):
            w = 1 << (k - 4)
            d = w >> 1
            while d >= 1:
                inter_pass(w, d)
                d >>= 1
            intra_pass(w)

        for r in range(ROWS_PER_W):
            row = row0 + r
            pltpu.sync_copy(comb_v.at[r], sorted_hbm.at[row])
            idx_v[...] = top_idx[r]
            pltpu.sync_copy(idx_v, idxpad_hbm.at[row])

    return pl.kernel(
        body,
        out_type=[
            jax.ShapeDtypeStruct((B, LANES), jnp.int32),
            jax.ShapeDtypeStruct((B, N), jnp.float32),
        ],
        mesh=plsc.VectorSubcoreMesh(core_axis_name="c", subcore_axis_name="s"),
        compiler_params=pltpu.CompilerParams(needs_layout_passes=False),
        scratch_types=[
            pltpu.VMEM((ROWS_PER_W, N), jnp.float32),
            pltpu.VMEM((ROWS_PER_W, N), jnp.float32),
            pltpu.VMEM((ROWS_PER_W, N), jnp.float32),
            pltpu.VMEM((ROWS_PER_W, N), jnp.float32),
            pltpu.VMEM((ROWS_PER_W, N), jnp.float32),
            pltpu.VMEM((LANES,), jnp.int32),
        ],
    )


def _tc_gather_body(idx_ref, cand_ref, out_ref, len_ref):
    b = pl.program_id(0)
    sub = idx_ref[b, 0] % 8
    row = cand_ref[0, pl.ds(sub, 1), :]
    out_ref[0] = row
    len_ref[0] = jnp.broadcast_to(
        jnp.sum((row != PAD_ID).astype(jnp.int32)), (1, L))


@functools.lru_cache(maxsize=1)
def _build_tc_gather():
    grid_spec = pltpu.PrefetchScalarGridSpec(
        num_scalar_prefetch=1,
        grid=(B,),
        in_specs=[
            pl.BlockSpec((1, 8, L), lambda b, idx_ref: (b, idx_ref[b, 0] // 8, 0)),
        ],
        out_specs=[
            pl.BlockSpec((1, 1, L), lambda b, idx_ref: (b, 0, 0)),
            pl.BlockSpec((1, 1, L), lambda b, idx_ref: (b, 0, 0)),
        ],
    )
    return pl.pallas_call(
        _tc_gather_body,
        grid_spec=grid_spec,
        out_shape=[
            jax.ShapeDtypeStruct((B, 1, L), jnp.int32),
            jax.ShapeDtypeStruct((B, 1, L), jnp.int32),
        ],
    )


def kernel(candidates, lengths, scores, ngram_scores, backtrans_scores,
           qa_scores):
    del lengths
    idxpad, sorted_scores = _build()(
        ngram_scores, backtrans_scores, scores, qa_scores)
    out3, len3 = _build_tc_gather()(idxpad, candidates)
    return out3[:, 0, :], len3[:, 0, 0], sorted_scores


# speedup vs baseline: 6.0672x; 1.4843x over previous
import functools

import jax
import jax.numpy as jnp
from jax import lax
from jax.experimental import pallas as pl
from jax.experimental.pallas import tpu as pltpu
from jax.experimental.pallas import tpu_sc as plsc

PAD_ID = 0
B, N, L = 64, 2048, 32
LANES = 16
V = N // LANES
HALF = V // 2
ROWS_PER_W = 2


def _vsort_desc(v):
    return lax.rev(jnp.sort(v), (0,))


@functools.lru_cache(maxsize=1)
def _build_sc_sort():
    info = plsc.get_sparse_core_info()
    nc = info.num_cores

    def body(ng_hbm, bt_hbm, nll_hbm, qa_hbm, sorted_hbm,
             ng_v, bt_v, nll_v, qa_v, comb_v, sem):
        wid = lax.axis_index("s") * nc + lax.axis_index("c")
        row0 = wid * ROWS_PER_W

        copies = []
        for r in range(ROWS_PER_W):
            row = row0 + r
            copies.append(pltpu.async_copy(ng_hbm.at[row], ng_v.at[r], sem))
            copies.append(pltpu.async_copy(bt_hbm.at[row], bt_v.at[r], sem))
            copies.append(pltpu.async_copy(nll_hbm.at[row], nll_v.at[r], sem))
            copies.append(pltpu.async_copy(qa_hbm.at[row], qa_v.at[r], sem))
        for c in copies:
            c.wait()

        def combine_body(j, _):
            for u in range(2):
                sl = pl.ds((j * 2 + u) * LANES, LANES)
                for r in range(ROWS_PER_W):
                    c = (ng_v[r, sl] * 1.5
                         + (bt_v[r, sl] + nll_v[r, sl]) * 0.5) \
                        * (qa_v[r, sl] * 0.9 + 0.1)
                    comb_v[r, sl] = _vsort_desc(c)
            return 0

        lax.fori_loop(0, V // 2, combine_body, 0)

        def mirror_pass(w):
            def pass_body(j, _):
                for u in range(2):
                    jj = j * 2 + u
                    t = jj // w
                    i = jj % w
                    a = 2 * w * t + i
                    b = 2 * w * t + (2 * w - 1 - i)
                    sa = pl.ds(a * LANES, LANES)
                    sb = pl.ds(b * LANES, LANES)
                    for r in range(ROWS_PER_W):
                        va = comb_v[r, sa]
                        vb = lax.rev(comb_v[r, sb], (0,))
                        comb_v[r, sa] = jnp.maximum(va, vb)
                        comb_v[r, sb] = lax.rev(jnp.minimum(va, vb), (0,))
                return 0

            lax.fori_loop(0, HALF // 2, pass_body, 0)

        def inner_pass(d):
            def pass_body(j, _):
                for u in range(2):
                    jj = j * 2 + u
                    a = (jj // d) * (2 * d) + (jj % d)
                    b = a + d
                    sa = pl.ds(a * LANES, LANES)
                    sb = pl.ds(b * LANES, LANES)
                    for r in range(ROWS_PER_W):
                        va = comb_v[r, sa]
                        vb = comb_v[r, sb]
                        comb_v[r, sa] = jnp.maximum(va, vb)
                        comb_v[r, sb] = jnp.minimum(va, vb)
                return 0

            lax.fori_loop(0, HALF // 2, pass_body, 0)

        def vsort_pass():
            def pass_body(j, _):
                for u in range(2):
                    sl = pl.ds((j * 2 + u) * LANES, LANES)
                    for r in range(ROWS_PER_W):
                        comb_v[r, sl] = _vsort_desc(comb_v[r, sl])
                return 0

            lax.fori_loop(0, V // 2, pass_body, 0)

        w = 1
        while w <= V // 2:
            mirror_pass(w)
            d = w // 2
            while d >= 1:
                inner_pass(d)
                d //= 2
            vsort_pass()
            w *= 2

        for r in range(ROWS_PER_W):
            pltpu.sync_copy(comb_v.at[r], sorted_hbm.at[row0 + r])

    return pl.kernel(
        body,
        out_type=jax.ShapeDtypeStruct((B, N), jnp.float32),
        mesh=plsc.VectorSubcoreMesh(core_axis_name="c", subcore_axis_name="s"),
        compiler_params=pltpu.CompilerParams(needs_layout_passes=False),
        scratch_types=[
            pltpu.VMEM((ROWS_PER_W, N), jnp.float32),
            pltpu.VMEM((ROWS_PER_W, N), jnp.float32),
            pltpu.VMEM((ROWS_PER_W, N), jnp.float32),
            pltpu.VMEM((ROWS_PER_W, N), jnp.float32),
            pltpu.VMEM((ROWS_PER_W, N), jnp.float32),
            pltpu.SemaphoreType.DMA,
        ],
    )


def _tc_argmax_body(ng_ref, bt_ref, nll_ref, qa_ref, idx_ref):
    comb = (ng_ref[...] * 1.5 + (bt_ref[...] + nll_ref[...]) * 0.5) \
        * (qa_ref[...] * 0.9 + 0.1)
    m = jnp.max(comb, axis=1, keepdims=True)
    iota = lax.broadcasted_iota(jnp.int32, (B, N), 1)
    idx_ref[...] = jnp.min(jnp.where(comb == m, iota, N), axis=1,
                           keepdims=True)


@functools.lru_cache(maxsize=1)
def _build_tc_argmax():
    return pl.pallas_call(
        _tc_argmax_body,
        out_shape=jax.ShapeDtypeStruct((B, 1), jnp.int32),
    )


def _tc_gather_body(idx_ref, cand_ref, out_ref, len_ref, sem):
    copies = [
        pltpu.make_async_copy(cand_ref.at[b, idx_ref[b, 0]], out_ref.at[b],
                              sem)
        for b in range(B)
    ]
    for c in copies:
        c.start()
    for c in copies:
        c.wait()
    vals = out_ref[...]
    len_ref[...] = jnp.sum((vals != PAD_ID).astype(jnp.int32), axis=1,
                           keepdims=True)


@functools.lru_cache(maxsize=1)
def _build_tc_gather():
    grid_spec = pltpu.PrefetchScalarGridSpec(
        num_scalar_prefetch=1,
        grid=(1,),
        in_specs=[pl.BlockSpec(memory_space=pl.ANY)],
        out_specs=[
            pl.BlockSpec((B, L), lambda g, idx_ref: (0, 0)),
            pl.BlockSpec((B, 1), lambda g, idx_ref: (0, 0)),
        ],
        scratch_shapes=[pltpu.SemaphoreType.DMA],
    )
    return pl.pallas_call(
        _tc_gather_body,
        grid_spec=grid_spec,
        out_shape=[
            jax.ShapeDtypeStruct((B, L), jnp.int32),
            jax.ShapeDtypeStruct((B, 1), jnp.int32),
        ],
    )


def kernel(candidates, lengths, scores, ngram_scores, backtrans_scores,
           qa_scores):
    del lengths
    sorted_scores = _build_sc_sort()(
        ngram_scores, backtrans_scores, scores, qa_scores)
    idx = _build_tc_argmax()(
        ngram_scores, backtrans_scores, scores, qa_scores)
    out, lens = _build_tc_gather()(idx, candidates)
    return out, lens[:, 0], sorted_scores
